# baseline (speedup 1.0000x reference)
import jax
import jax.numpy as jnp
from jax.experimental import pallas as pl
from jax.experimental.pallas import tpu as pltpu

B = 128
N = 100000
BN = 12800
K = (N + BN - 1) // BN


def _body(x_ref, idx_out, val_out, best_val, best_idx):
    s = pl.program_id(0)

    @pl.when(s == 0)
    def _init():
        best_val[...] = jnp.full((B, 1), -jnp.inf, jnp.float32)
        best_idx[...] = jnp.zeros((B, 1), jnp.int32)

    col = jax.lax.broadcasted_iota(jnp.int32, (B, BN), 1) + s * BN
    x = jnp.where(col < N, x_ref[...], -jnp.inf)
    lmax = jnp.max(x, axis=-1, keepdims=True)
    cand = jnp.where(x == lmax, col, jnp.int32(2**31 - 1))
    lidx = jnp.min(cand, axis=-1, keepdims=True)

    better = lmax > best_val[...]
    best_val[...] = jnp.where(better, lmax, best_val[...])
    best_idx[...] = jnp.where(better, lidx, best_idx[...])

    @pl.when(s == K - 1)
    def _done():
        idx_out[...] = best_idx[...]
        val_out[...] = best_val[...]


def kernel(logits):
    idx, val = pl.pallas_call(
        _body,
        grid=(K,),
        in_specs=[pl.BlockSpec((B, BN), lambda s: (0, s))],
        out_specs=[
            pl.BlockSpec((B, 1), lambda s: (0, 0)),
            pl.BlockSpec((B, 1), lambda s: (0, 0)),
        ],
        out_shape=[
            jax.ShapeDtypeStruct((B, 1), jnp.int32),
            jax.ShapeDtypeStruct((B, 1), jnp.float32),
        ],
        scratch_shapes=[
            pltpu.VMEM((B, 1), jnp.float32),
            pltpu.VMEM((B, 1), jnp.int32),
        ],
    )(logits)
    return idx[:, 0], val
